# baseline (speedup 1.0000x reference)
import jax
import jax.numpy as jnp
from jax import lax
from jax.experimental import pallas as pl
from jax.experimental.pallas import tpu as pltpu
from jax.experimental.pallas import tpu_sc as plsc

VOCAB = 1000000
EMBED_DIM = 64
PAD_DIM = 128
BATCH = 16384
HIST = 50

_NC = 2
_NS = 16
_NW = _NC * _NS

_TOTAL = BATCH * HIST
_PER_W = _TOTAL // _NW
_GROUP = 128
_NG = _PER_W // _GROUP
_NBUF = 5
_AHEAD = 4


def _body(idx_hbm, tab_hbm, out_hbm, idx_v,
          rows0, rows1, rows2, rows3, rows4,
          gsem0, gsem1, gsem2, gsem3, gsem4,
          ssem0, ssem1, ssem2, ssem3, ssem4):
    wid = lax.axis_index("s") * _NC + lax.axis_index("c")
    base = wid * _PER_W
    pltpu.sync_copy(idx_hbm.at[pl.ds(base, _PER_W)], idx_v)

    rows = (rows0, rows1, rows2, rows3, rows4)
    gsems = (gsem0, gsem1, gsem2, gsem3, gsem4)
    ssems = (ssem0, ssem1, ssem2, ssem3, ssem4)

    def store_done_wait(slot):
        pltpu.make_async_copy(
            rows[slot].at[:, pl.ds(0, EMBED_DIM)],
            out_hbm.at[pl.ds(base, _GROUP)],
            ssems[slot]).wait()

    def retire(slot, g):
        pltpu.make_async_copy(
            tab_hbm.at[pl.ds(0, _GROUP)], rows[slot], gsems[slot]).wait()
        pltpu.async_copy(
            rows[slot].at[:, pl.ds(0, EMBED_DIM)],
            out_hbm.at[pl.ds(base + g * _GROUP, _GROUP)],
            ssems[slot])

    def step(t, _):
        for slot in range(_NBUF):
            g = t * _NBUF + slot
            @pl.when(t > 0)
            def _():
                store_done_wait(slot)

            pltpu.async_copy(
                tab_hbm.at[idx_v.at[pl.ds(g * _GROUP, _GROUP)]],
                rows[slot], gsems[slot])

            ps = (slot - _AHEAD) % _NBUF
            if slot >= _AHEAD:
                retire(ps, g - _AHEAD)
            else:
                @pl.when(t > 0)
                def _():
                    retire(ps, g - _AHEAD)
        return ()

    lax.fori_loop(0, _NG // _NBUF, step, (), unroll=False)
    for g in range(_NG - _AHEAD, _NG):
        retire(g % _NBUF, g)
    for slot in range(_NBUF):
        store_done_wait(slot)


@jax.jit
def _lookup(idx_flat, tab_padded):
    mesh = plsc.VectorSubcoreMesh(core_axis_name="c", subcore_axis_name="s")
    fn = pl.kernel(
        _body,
        out_type=jax.ShapeDtypeStruct((_TOTAL, EMBED_DIM), jnp.float32),
        mesh=mesh,
        scratch_types=(
            [pltpu.VMEM((_PER_W,), jnp.int32)]
            + [pltpu.VMEM((_GROUP, PAD_DIM), jnp.float32)] * _NBUF
            + [pltpu.SemaphoreType.DMA] * (2 * _NBUF)
        ),
        compiler_params=pltpu.CompilerParams(use_tc_tiling_on_sc=False),
    )
    return fn(idx_flat, tab_padded)


def kernel(input_vec, word_embedding):
    idx_flat = input_vec.reshape(-1).astype(jnp.int32)
    tab_padded = jnp.pad(word_embedding, ((0, 0), (0, PAD_DIM - EMBED_DIM)))
    out = _lookup(idx_flat, tab_padded)
    return out.reshape(BATCH, HIST, EMBED_DIM)


# speedup vs baseline: 1.1597x; 1.1597x over previous
import jax
import jax.numpy as jnp
from jax import lax
from jax.experimental.layout import Format, Layout, with_layout_constraint
from jax.experimental import pallas as pl
from jax.experimental.pallas import tpu as pltpu
from jax.experimental.pallas import tpu_sc as plsc

VOCAB = 1000000
EMBED_DIM = 64
PAD_DIM = 128
BATCH = 16384
HIST = 50

_NC = 2
_NS = 16
_NW = _NC * _NS

_TOTAL = BATCH * HIST
_PER_W = _TOTAL // _NW
_GROUP = 128
_NG = _PER_W // _GROUP
_NBUF = 5
_AHEAD = 4


def _body(idx_hbm, tab_hbm, out_hbm, idx_v,
          rows0, rows1, rows2, rows3, rows4,
          gsem0, gsem1, gsem2, gsem3, gsem4,
          ssem0, ssem1, ssem2, ssem3, ssem4):
    wid = lax.axis_index("s") * _NC + lax.axis_index("c")
    base = wid * _PER_W
    pltpu.sync_copy(idx_hbm.at[pl.ds(base, _PER_W)], idx_v)

    rows = (rows0, rows1, rows2, rows3, rows4)
    gsems = (gsem0, gsem1, gsem2, gsem3, gsem4)
    ssems = (ssem0, ssem1, ssem2, ssem3, ssem4)

    def store_done_wait(slot):
        pltpu.make_async_copy(
            rows[slot].at[:, pl.ds(0, EMBED_DIM)],
            out_hbm.at[pl.ds(base, _GROUP)],
            ssems[slot]).wait()

    def retire(slot, g):
        pltpu.make_async_copy(
            tab_hbm.at[pl.ds(0, _GROUP)], rows[slot], gsems[slot]).wait()
        pltpu.async_copy(
            rows[slot].at[:, pl.ds(0, EMBED_DIM)],
            out_hbm.at[pl.ds(base + g * _GROUP, _GROUP)],
            ssems[slot])

    def step(t, _):
        for slot in range(_NBUF):
            g = t * _NBUF + slot
            @pl.when(t > 0)
            def _():
                store_done_wait(slot)

            pltpu.async_copy(
                tab_hbm.at[idx_v.at[pl.ds(g * _GROUP, _GROUP)]],
                rows[slot], gsems[slot])

            ps = (slot - _AHEAD) % _NBUF
            if slot >= _AHEAD:
                retire(ps, g - _AHEAD)
            else:
                @pl.when(t > 0)
                def _():
                    retire(ps, g - _AHEAD)
        return ()

    lax.fori_loop(0, _NG // _NBUF, step, (), unroll=False)
    for g in range(_NG - _AHEAD, _NG):
        retire(g % _NBUF, g)
    for slot in range(_NBUF):
        store_done_wait(slot)


@jax.jit
def _lookup(idx_flat, tab_padded):
    mesh = plsc.VectorSubcoreMesh(core_axis_name="c", subcore_axis_name="s")
    fn = pl.kernel(
        _body,
        out_type=jax.ShapeDtypeStruct((_TOTAL, EMBED_DIM), jnp.float32),
        mesh=mesh,
        scratch_types=(
            [pltpu.VMEM((_PER_W,), jnp.int32)]
            + [pltpu.VMEM((_GROUP, PAD_DIM), jnp.float32)] * _NBUF
            + [pltpu.SemaphoreType.DMA] * (2 * _NBUF)
        ),
        compiler_params=pltpu.CompilerParams(use_tc_tiling_on_sc=False),
    )
    return fn(idx_flat, tab_padded)


def kernel(input_vec, word_embedding):
    idx_flat = input_vec.reshape(-1).astype(jnp.int32)
    tab_padded = jnp.pad(word_embedding, ((0, 0), (0, PAD_DIM - EMBED_DIM)))
    out = _lookup(idx_flat, tab_padded)
    out3 = out.reshape(BATCH, HIST, EMBED_DIM)
    return with_layout_constraint(out3, Layout(major_to_minor=(0, 1, 2)))
